# baseline (speedup 1.0000x reference)
import functools

import jax
import jax.numpy as jnp
from jax import lax
from jax.experimental import pallas as pl
from jax.experimental.pallas import tpu as pltpu
from jax.experimental.pallas import tpu_sc as plsc

_LB = 128


@functools.lru_cache(maxsize=None)
def _make_tc_transpose(v: int, d: int, cols: int):
    grid = (v + cols - 1) // cols

    def body(x_ref, o_ref):
        o_ref[:, 0:d] = x_ref[...].T

    return pl.pallas_call(
        body,
        grid=(grid,),
        in_specs=[pl.BlockSpec((d, cols), lambda i: (0, i))],
        out_specs=pl.BlockSpec((cols, 2 * d), lambda i: (i, 0)),
        out_shape=jax.ShapeDtypeStruct((v, 2 * d), jnp.float32),
    )


@functools.lru_cache(maxsize=None)
def _make_gather(hist: int, batch: int, d: int, v: int):
    info = plsc.get_sparse_core_info()
    nw = info.num_cores * info.num_subcores
    n_blocks = hist * (batch // _LB)
    assert n_blocks % (2 * nw) == 0
    blk_per_w = n_blocks // nw
    bc_per_h = batch // _LB

    mesh = plsc.VectorSubcoreMesh(core_axis_name="c", subcore_axis_name="s")

    @functools.partial(
        pl.kernel,
        mesh=mesh,
        out_type=jax.ShapeDtypeStruct((hist, d, batch), jnp.float32),
        scratch_types=[
            pltpu.VMEM((2, _LB), jnp.int32),
            pltpu.VMEM((2, _LB, 2 * d + 1), jnp.float32),
            pltpu.VMEM((2, d, _LB), jnp.float32),
            pltpu.SemaphoreType.DMA,
            pltpu.SemaphoreType.DMA,
            pltpu.SemaphoreType.DMA,
        ],
        compiler_params=pltpu.CompilerParams(use_tc_tiling_on_sc=True,
                                             needs_layout_passes=False,
                                             disable_bounds_checks=True),
    )
    def gather(tbl_hbm, xt_hbm, out_hbm, idx_v, g_v, b_v,
               sem_i, sem_g, sem_o):
        wid = lax.axis_index("s") * info.num_cores + lax.axis_index("c")
        n0 = wid * blk_per_w
        vjs = [lax.iota(jnp.int32, 16) + jg * 16 for jg in range(8)]

        def hb(n_loc):
            n = n0 + n_loc
            return n // bc_per_h, (n % bc_per_h) * _LB

        def load_idx(n_loc, bu):
            h, b0 = hb(n_loc)
            pltpu.async_copy(xt_hbm.at[h, pl.ds(b0, _LB)], idx_v.at[bu],
                             sem_i)

        def wait_idx(bu):
            pltpu.make_async_copy(xt_hbm.at[0, pl.ds(0, _LB)], idx_v.at[bu],
                                  sem_i).wait()

        def fire(bu):
            pltpu.async_copy(tbl_hbm.at[idx_v.at[bu]],
                             g_v.at[bu, :, pl.ds(0, 2 * d)], sem_g)

        def wait_gather(bu):
            pltpu.make_async_copy(tbl_hbm.at[pl.ds(0, _LB)],
                                  g_v.at[bu, :, pl.ds(0, 2 * d)],
                                  sem_g).wait()

        def transpose(bu):
            g_ref = g_v.at[bu]
            b_ref = b_v.at[bu]

            @plsc.parallel_loop(0, d, unroll=8)
            def dbody(dd):
                vd = lax.broadcast(dd, (16,))
                for jg in range(8):
                    vals = plsc.load_gather(g_ref, [vjs[jg], vd])
                    b_ref[dd, pl.ds(jg * 16, 16)] = vals

        def writeback(n_loc, bu):
            h, b0 = hb(n_loc)
            for dr in range(d // 8):
                pltpu.async_copy(
                    b_v.at[bu, pl.ds(dr * 8, 8)],
                    out_hbm.at[h, pl.ds(dr * 8, 8), pl.ds(b0, _LB)],
                    sem_o,
                )

        def drain_writeback():
            pltpu.make_async_copy(b_v.at[0],
                                  out_hbm.at[0, pl.ds(0, d), pl.ds(0, _LB)],
                                  sem_o).wait()

        load_idx(0, 0)
        wait_idx(0)
        fire(0)
        load_idx(1, 1)

        def body(m, carry):
            for j in (0, 1):
                bu = j
                n_loc = 2 * m + j
                if j == 0:
                    wait_idx(1)
                    fire(1)
                else:
                    @pl.when(m < blk_per_w // 2 - 1)
                    def _():
                        wait_idx(0)
                        fire(0)
                @pl.when(m < blk_per_w // 2 - 1)
                def _():
                    load_idx(n_loc + 2, bu)
                @pl.when(m > 0)
                def _():
                    drain_writeback()
                wait_gather(bu)
                transpose(bu)
                writeback(n_loc, bu)
            return carry

        lax.fori_loop(0, blk_per_w // 2, body, 0, unroll=False)
        drain_writeback()
        drain_writeback()

    return gather


def kernel(x, emb_loc):
    b, h = x.shape
    v, d = emb_loc.shape
    xt = x.T
    tbl = _make_tc_transpose(v, d, 4096)(emb_loc.T)
    out3 = _make_gather(h, b, d, v)(tbl, xt)
    return out3.transpose(2, 0, 1)


# speedup vs baseline: 1.0149x; 1.0149x over previous
import functools

import jax
import jax.numpy as jnp
from jax import lax
from jax.experimental import pallas as pl
from jax.experimental.pallas import tpu as pltpu
from jax.experimental.pallas import tpu_sc as plsc

_LB = 256


@functools.lru_cache(maxsize=None)
def _make_tc_transpose(v: int, d: int, cols: int):
    grid = (v + cols - 1) // cols

    def body(x_ref, o_ref):
        o_ref[:, 0:d] = x_ref[...].T

    return pl.pallas_call(
        body,
        grid=(grid,),
        in_specs=[pl.BlockSpec((d, cols), lambda i: (0, i))],
        out_specs=pl.BlockSpec((cols, 2 * d), lambda i: (i, 0)),
        out_shape=jax.ShapeDtypeStruct((v, 2 * d), jnp.float32),
    )


@functools.lru_cache(maxsize=None)
def _make_gather(hist: int, batch: int, d: int, v: int):
    info = plsc.get_sparse_core_info()
    nw = info.num_cores * info.num_subcores
    n_blocks = hist * (batch // _LB)
    assert n_blocks % (2 * nw) == 0
    blk_per_w = n_blocks // nw
    bc_per_h = batch // _LB

    mesh = plsc.VectorSubcoreMesh(core_axis_name="c", subcore_axis_name="s")

    @functools.partial(
        pl.kernel,
        mesh=mesh,
        out_type=jax.ShapeDtypeStruct((hist, d, batch), jnp.float32),
        scratch_types=[
            pltpu.VMEM((2, 2, _LB // 2), jnp.int32),
            pltpu.VMEM((2, _LB, 2 * d), jnp.float32),
            pltpu.VMEM((2, d, _LB), jnp.float32),
            pltpu.SemaphoreType.DMA,
            pltpu.SemaphoreType.DMA,
            pltpu.SemaphoreType.DMA,
        ],
        compiler_params=pltpu.CompilerParams(use_tc_tiling_on_sc=True,
                                             needs_layout_passes=False,
                                             disable_bounds_checks=True),
    )
    def gather(tbl_hbm, xt_hbm, out_hbm, idx_v, g_v, b_v,
               sem_i, sem_g, sem_o):
        wid = lax.axis_index("s") * info.num_cores + lax.axis_index("c")
        n0 = wid * blk_per_w
        vjs = [lax.iota(jnp.int32, 16) + jg * 16 for jg in range(_LB // 16)]

        def hb(n_loc):
            n = n0 + n_loc
            return n // bc_per_h, (n % bc_per_h) * _LB

        def load_idx(n_loc, bu):
            h, b0 = hb(n_loc)
            for k in (0, 1):
                pltpu.async_copy(xt_hbm.at[h, pl.ds(b0 + k * (_LB // 2),
                                                    _LB // 2)],
                                 idx_v.at[bu, k], sem_i)

        def wait_idx(bu):
            pltpu.make_async_copy(xt_hbm.at[pl.ds(0, 2), pl.ds(0, _LB // 2)],
                                  idx_v.at[bu], sem_i).wait()

        def fire(bu):
            for k in (0, 1):
                pltpu.async_copy(tbl_hbm.at[idx_v.at[bu, k]],
                                 g_v.at[bu, pl.ds(k * (_LB // 2), _LB // 2)],
                                 sem_g)

        def wait_gather(bu):
            pltpu.make_async_copy(tbl_hbm.at[pl.ds(0, _LB)], g_v.at[bu],
                                  sem_g).wait()

        def transpose(bu):
            g_ref = g_v.at[bu]
            b_ref = b_v.at[bu]

            @plsc.parallel_loop(0, d, unroll=8)
            def dbody(dd):
                vd = lax.broadcast(dd, (16,))
                for jg in range(_LB // 16):
                    vals = plsc.load_gather(g_ref, [vjs[jg], vd])
                    b_ref[dd, pl.ds(jg * 16, 16)] = vals

        def writeback(n_loc, bu):
            h, b0 = hb(n_loc)
            for dr in range(d // 8):
                pltpu.async_copy(
                    b_v.at[bu, pl.ds(dr * 8, 8)],
                    out_hbm.at[h, pl.ds(dr * 8, 8), pl.ds(b0, _LB)],
                    sem_o,
                )

        def drain_writeback():
            pltpu.make_async_copy(b_v.at[0],
                                  out_hbm.at[0, pl.ds(0, d), pl.ds(0, _LB)],
                                  sem_o).wait()

        load_idx(0, 0)
        wait_idx(0)
        fire(0)
        load_idx(1, 1)

        def body(m, carry):
            for j in (0, 1):
                bu = j
                n_loc = 2 * m + j
                if j == 0:
                    wait_idx(1)
                    fire(1)
                else:
                    @pl.when(m < blk_per_w // 2 - 1)
                    def _():
                        wait_idx(0)
                        fire(0)
                @pl.when(m > 0)
                def _():
                    drain_writeback()
                wait_gather(bu)
                @pl.when(m < blk_per_w // 2 - 1)
                def _():
                    load_idx(n_loc + 2, bu)
                transpose(bu)
                writeback(n_loc, bu)
            return carry

        lax.fori_loop(0, blk_per_w // 2, body, 0, unroll=False)
        drain_writeback()
        drain_writeback()

    return gather


def kernel(x, emb_loc):
    b, h = x.shape
    v, d = emb_loc.shape
    xt = x.T
    tbl = _make_tc_transpose(v, d, 4096)(emb_loc.T)
    out3 = _make_gather(h, b, d, v)(tbl, xt)
    return out3.transpose(2, 0, 1)
